# Initial kernel scaffold; baseline (speedup 1.0000x reference)
import jax
import jax.numpy as jnp
from jax.experimental import pallas as pl


def kernel(positions, celltype, cadherin, radius):
    raise NotImplementedError("write your pallas kernel here")


# speedup vs baseline: 638.8607x; 638.8607x over previous
import functools

import jax
import jax.numpy as jnp
from jax.experimental import pallas as pl

BOX = 10.0
ALPHA = 2.8
R_ONSET = 1.7
R_CUTOFF = 2.0


def _tile_kernel(pr_ref, pc_ref, ctr_ref, ctc_ref, cad_ref, rrow_ref,
                 rcol_ref, out_ref, *, tr, tc):
    i = pl.program_id(0)
    j = pl.program_id(1)

    sig8 = rcol_ref[...] + rrow_ref[...]
    eps8 = cad_ref[...]
    e_sig = jnp.exp(ALPHA * sig8)
    a8 = eps8 * e_sig * e_sig
    b8 = 2.0 * eps8 * e_sig

    def onehot(ct):
        n = ct.shape[0]
        mx = jnp.max(ct, axis=1, keepdims=True)
        iota = jax.lax.broadcasted_iota(jnp.int32, (n, 8), 1)
        idx = jnp.min(jnp.where(ct == mx, iota, 8), axis=1, keepdims=True)
        idx = jnp.where(jnp.sum(ct, axis=1, keepdims=True) > 0.0, idx, 0)
        return (iota == idx).astype(jnp.float32)

    ohr = onehot(ctr_ref[...])
    ohc = onehot(ctc_ref[...])

    arow = jnp.dot(ohr, a8, preferred_element_type=jnp.float32)
    brow = jnp.dot(ohr, b8, preferred_element_type=jnp.float32)
    dimn = (((1,), (1,)), ((), ()))
    a_t = jax.lax.dot_general(arow, ohc, dimn,
                              preferred_element_type=jnp.float32)
    b_t = jax.lax.dot_general(brow, ohc, dimn,
                              preferred_element_type=jnp.float32)

    pr = pr_ref[...]
    pc = pc_ref[...]
    dr2 = jnp.zeros((tr, tc), jnp.float32)
    for k in range(3):
        d = pr[:, k:k + 1] - pc[k:k + 1, :]
        d = d - BOX * jnp.round(d * (1.0 / BOX))
        dr2 = dr2 + d * d

    safe = jnp.where(dr2 > 0.0, dr2, 1.0)
    r = jnp.sqrt(safe)
    t = jnp.exp(-ALPHA * r)
    u = (a_t * t - b_t) * t

    r2 = r * r
    r_on2 = R_ONSET * R_ONSET
    r_c2 = R_CUTOFF * R_CUTOFF
    inv_den = 1.0 / (r_c2 - r_on2) ** 3
    mid = ((r_c2 - r2) ** 2 * (r_c2 + 2.0 * r2 - 3.0 * r_on2)) * inv_den
    s = jnp.where(r < R_ONSET, 1.0,
                  jnp.where(r < R_CUTOFF, mid, 0.0))
    u = u * s

    rid = i * tr + jax.lax.broadcasted_iota(jnp.int32, (tr, tc), 0)
    cid = j * tc + jax.lax.broadcasted_iota(jnp.int32, (tr, tc), 1)
    u = jnp.where(rid == cid, 0.0, u)

    @pl.when((i == 0) & (j == 0))
    def _():
        out_ref[...] = jnp.zeros((1, 1), jnp.float32)

    out_ref[...] += 0.5 * jnp.sum(u).reshape(1, 1)


def kernel(positions, celltype, cadherin, radius):
    n = positions.shape[0]
    tr, tc = 256, 512
    nr, nc = n // tr, n // tc

    pos_row = jnp.pad(positions, ((0, 0), (0, 5)))
    pos_col = pos_row.T
    cad8 = jnp.reshape(cadherin, (8, 8))
    r8_row = jnp.reshape(radius[:8, 0], (1, 8))
    r8_col = jnp.reshape(radius[:8, 0], (8, 1))

    out = pl.pallas_call(
        functools.partial(_tile_kernel, tr=tr, tc=tc),
        grid=(nr, nc),
        in_specs=[
            pl.BlockSpec((tr, 8), lambda i, j: (i, 0)),
            pl.BlockSpec((8, tc), lambda i, j: (0, j)),
            pl.BlockSpec((tr, 8), lambda i, j: (i, 0)),
            pl.BlockSpec((tc, 8), lambda i, j: (j, 0)),
            pl.BlockSpec((8, 8), lambda i, j: (0, 0)),
            pl.BlockSpec((1, 8), lambda i, j: (0, 0)),
            pl.BlockSpec((8, 1), lambda i, j: (0, 0)),
        ],
        out_specs=pl.BlockSpec((1, 1), lambda i, j: (0, 0)),
        out_shape=jax.ShapeDtypeStruct((1, 1), jnp.float32),
    )(pos_row, pos_col, celltype, celltype, cad8, r8_row, r8_col)
    return jnp.reshape(out, ())


# speedup vs baseline: 888.5447x; 1.3908x over previous
import functools

import jax
import jax.numpy as jnp
from jax.experimental import pallas as pl

BOX = 10.0
ALPHA = 2.8
R_ONSET = 1.7
R_CUTOFF = 2.0


def _prologue_kernel(ct_ref, ctt_ref, cad_ref, rrow_ref, rcol_ref,
                     arow_ref, brow_ref, oht_ref):
    sig8 = rcol_ref[...] + rrow_ref[...]
    eps8 = cad_ref[...]
    e_sig = jnp.exp(ALPHA * sig8)
    a8 = eps8 * e_sig * e_sig
    b8 = 2.0 * eps8 * e_sig

    ct = ct_ref[...]
    n = ct.shape[0]
    mx = jnp.max(ct, axis=1, keepdims=True)
    iota = jax.lax.broadcasted_iota(jnp.int32, (n, 8), 1)
    idx = jnp.min(jnp.where(ct == mx, iota, 8), axis=1, keepdims=True)
    idx = jnp.where(jnp.sum(ct, axis=1, keepdims=True) > 0.0, idx, 0)
    oh = (iota == idx).astype(jnp.float32)
    arow_ref[...] = jnp.dot(oh, a8, preferred_element_type=jnp.float32)
    brow_ref[...] = jnp.dot(oh, b8, preferred_element_type=jnp.float32)

    ctt = ctt_ref[...]
    mxt = jnp.max(ctt, axis=0, keepdims=True)
    iota_t = jax.lax.broadcasted_iota(jnp.int32, (8, n), 0)
    idx_t = jnp.min(jnp.where(ctt == mxt, iota_t, 8), axis=0, keepdims=True)
    idx_t = jnp.where(jnp.sum(ctt, axis=0, keepdims=True) > 0.0, idx_t, 0)
    oht_ref[...] = (iota_t == idx_t).astype(jnp.float32)


def _tile_kernel(pr_ref, pc_ref, arow_ref, brow_ref, oht_ref, out_ref,
                 *, tr, tc):
    i = pl.program_id(0)
    j = pl.program_id(1)

    oht = oht_ref[...]
    a_t = jnp.dot(arow_ref[...], oht, preferred_element_type=jnp.float32)
    b_t = jnp.dot(brow_ref[...], oht, preferred_element_type=jnp.float32)

    pr = pr_ref[...]
    pc = pc_ref[...]
    dr2 = jnp.zeros((tr, tc), jnp.float32)
    for k in range(3):
        d = pr[:, k:k + 1] - pc[k:k + 1, :]
        d = d - BOX * jnp.round(d * (1.0 / BOX))
        dr2 = dr2 + d * d

    safe = jnp.where(dr2 > 0.0, dr2, 1.0)
    r = jnp.sqrt(safe)
    t = jnp.exp(-ALPHA * r)

    r_on2 = R_ONSET * R_ONSET
    r_c2 = R_CUTOFF * R_CUTOFF
    inv_den = 1.0 / (r_c2 - r_on2) ** 3
    x = jnp.minimum(jnp.maximum(r * r, r_on2), r_c2)
    s = ((r_c2 - x) ** 2 * (r_c2 + 2.0 * x - 3.0 * r_on2)) * inv_den

    u = (a_t * t - b_t) * (t * s)

    @pl.when((i == 0) & (j == 0))
    def _():
        out_ref[...] = jnp.zeros((1, 1), jnp.float32)

    @pl.when((i * tr < (j + 1) * tc) & (j * tc < (i + 1) * tr))
    def _():
        rid = i * tr + jax.lax.broadcasted_iota(jnp.int32, (tr, tc), 0)
        cid = j * tc + jax.lax.broadcasted_iota(jnp.int32, (tr, tc), 1)
        diag = jnp.sum(jnp.where(rid == cid, u, 0.0))
        out_ref[...] -= 0.5 * diag.reshape(1, 1)

    out_ref[...] += 0.5 * jnp.sum(u).reshape(1, 1)


def kernel(positions, celltype, cadherin, radius):
    n = positions.shape[0]
    tr, tc = 256, 512
    nr, nc = n // tr, n // tc

    pos_row = jnp.pad(positions, ((0, 0), (0, 5)))
    pos_col = pos_row.T
    cad8 = jnp.reshape(cadherin, (8, 8))
    r8_row = jnp.reshape(radius[:8, 0], (1, 8))
    r8_col = jnp.reshape(radius[:8, 0], (8, 1))

    arow, brow, oht = pl.pallas_call(
        _prologue_kernel,
        out_shape=[
            jax.ShapeDtypeStruct((n, 8), jnp.float32),
            jax.ShapeDtypeStruct((n, 8), jnp.float32),
            jax.ShapeDtypeStruct((8, n), jnp.float32),
        ],
    )(celltype, celltype.T, cad8, r8_row, r8_col)

    out = pl.pallas_call(
        functools.partial(_tile_kernel, tr=tr, tc=tc),
        grid=(nr, nc),
        in_specs=[
            pl.BlockSpec((tr, 8), lambda i, j: (i, 0)),
            pl.BlockSpec((8, tc), lambda i, j: (0, j)),
            pl.BlockSpec((tr, 8), lambda i, j: (i, 0)),
            pl.BlockSpec((tr, 8), lambda i, j: (i, 0)),
            pl.BlockSpec((8, tc), lambda i, j: (0, j)),
        ],
        out_specs=pl.BlockSpec((1, 1), lambda i, j: (0, 0)),
        out_shape=jax.ShapeDtypeStruct((1, 1), jnp.float32),
    )(pos_row, pos_col, arow, brow, oht)
    return jnp.reshape(out, ())
